# Initial kernel scaffold; baseline (speedup 1.0000x reference)
import jax
import jax.numpy as jnp
from jax.experimental import pallas as pl


def kernel(x, weight):
    raise NotImplementedError("write your pallas kernel here")


# speedup vs baseline: 1.1595x; 1.1595x over previous
import jax
import jax.numpy as jnp
from jax import lax
from jax.experimental import pallas as pl
from jax.experimental.pallas import tpu as pltpu
from jax.experimental.pallas import tpu_sc as plsc

D = 128
L = 16

_info = plsc.get_sparse_core_info()
NC, NS = _info.num_cores, _info.num_subcores
NW = NC * NS
NCHUNK = 128 // NW
CROWS = 128


def _sc_body(x_hbm, w_hbm, out_hbm, x_v, rows_v, *sems):
    wid = lax.axis_index("s") * NC + lax.axis_index("c")

    pltpu.sync_copy(x_hbm, x_v)

    gathers = [
        pltpu.async_copy(
            w_hbm.at[x_v.at[wid * NCHUNK + j]],
            rows_v.at[j],
            sems[j],
        )
        for j in range(NCHUNK)
    ]

    outs = []
    for j in range(NCHUNK):
        gathers[j].wait()

        def add_rows(l, carry, j=j):
            for t in range(D // L):
                sl = pl.ds(t * L, L)
                a = x_v[l, sl].astype(jnp.float32)
                rows_v[j, l, sl] = rows_v[j, l, sl] + a
            return carry

        lax.fori_loop(0, CROWS, add_rows, 0)

        outs.append(
            pltpu.async_copy(
                rows_v.at[j],
                out_hbm.at[wid * NCHUNK + j],
                sems[NCHUNK + j],
            )
        )

    for c in outs:
        c.wait()


@jax.jit
def _embed(x, weight):
    mesh = plsc.VectorSubcoreMesh(core_axis_name="c", subcore_axis_name="s")
    fn = pl.kernel(
        _sc_body,
        out_type=jax.ShapeDtypeStruct((128, CROWS, D), jnp.float32),
        mesh=mesh,
        scratch_types=[
            pltpu.VMEM((128, 128), jnp.int32),
            pltpu.VMEM((NCHUNK, CROWS, D), jnp.float32),
        ]
        + [pltpu.SemaphoreType.DMA] * (2 * NCHUNK),
    )
    return fn(x, weight)


def kernel(x, weight):
    return _embed(x, weight)


# speedup vs baseline: 1.2846x; 1.1079x over previous
import jax
import jax.numpy as jnp
from jax import lax
from jax.experimental import pallas as pl
from jax.experimental.pallas import tpu as pltpu
from jax.experimental.pallas import tpu_sc as plsc

D = 128
L = 16

_info = plsc.get_sparse_core_info()
NC, NS = _info.num_cores, _info.num_subcores
NW = NC * NS
NCHUNK = 128 // NW
CROWS = 128


def _sc_body(x_hbm, w_hbm, out_hbm, x_v, rows_v, *sems):
    wid = lax.axis_index("s") * NC + lax.axis_index("c")

    pltpu.sync_copy(x_hbm, x_v)

    gathers = []
    for j in range(NCHUNK):

        def fill_rows(l, carry, j=j):
            for t in range(D // L):
                sl = pl.ds(t * L, L)
                rows_v[j, l, sl] = x_v[l, sl].astype(jnp.float32)
            return carry

        lax.fori_loop(0, CROWS, fill_rows, 0)

        gathers.append(
            pltpu.async_copy(
                w_hbm.at[x_v.at[wid * NCHUNK + j]],
                rows_v.at[j],
                sems[j],
                add=True,
            )
        )

    outs = []
    for j in range(NCHUNK):
        gathers[j].wait()
        outs.append(
            pltpu.async_copy(
                rows_v.at[j],
                out_hbm.at[wid * NCHUNK + j],
                sems[NCHUNK + j],
            )
        )

    for c in outs:
        c.wait()


@jax.jit
def _embed(x, weight):
    mesh = plsc.VectorSubcoreMesh(core_axis_name="c", subcore_axis_name="s")
    fn = pl.kernel(
        _sc_body,
        out_type=jax.ShapeDtypeStruct((128, CROWS, D), jnp.float32),
        mesh=mesh,
        scratch_types=[
            pltpu.VMEM((128, 128), jnp.int32),
            pltpu.VMEM((NCHUNK, CROWS, D), jnp.float32),
        ]
        + [pltpu.SemaphoreType.DMA] * (2 * NCHUNK),
    )
    return fn(x, weight)


def kernel(x, weight):
    return _embed(x, weight)


# speedup vs baseline: 1.2926x; 1.0062x over previous
import jax
import jax.numpy as jnp
from jax import lax
from jax.experimental import pallas as pl
from jax.experimental.pallas import tpu as pltpu
from jax.experimental.pallas import tpu_sc as plsc

D = 128
L = 16

_info = plsc.get_sparse_core_info()
NC, NS = _info.num_cores, _info.num_subcores
NW = NC * NS
NCHUNK = 128 // NW
CROWS = 128
SUB = 2


def _sc_body(x_hbm, w_hbm, out_hbm, x_v, rows_v, *sems):
    wid = lax.axis_index("s") * NC + lax.axis_index("c")

    pltpu.sync_copy(x_hbm, x_v)

    NSTG = NCHUNK * SUB
    gathers = []
    for g in range(NSTG):
        j, s = divmod(g, SUB)
        base = s * (CROWS // SUB)

        def fill_rows(l, carry, j=j, base=base):
            for t in range(D // L):
                sl = pl.ds(t * L, L)
                rows_v[j, base + l, sl] = x_v[base + l, sl].astype(jnp.float32)
            return carry

        lax.fori_loop(0, CROWS // SUB, fill_rows, 0)

        gathers.append(
            pltpu.async_copy(
                w_hbm.at[x_v.at[wid * NCHUNK + j, pl.ds(base, CROWS // SUB)]],
                rows_v.at[j, pl.ds(base, CROWS // SUB)],
                sems[g],
                add=True,
            )
        )

    outs = []
    for g in range(NSTG):
        j, s = divmod(g, SUB)
        base = s * (CROWS // SUB)
        gathers[g].wait()
        outs.append(
            pltpu.async_copy(
                rows_v.at[j, pl.ds(base, CROWS // SUB)],
                out_hbm.at[wid * NCHUNK + j, pl.ds(base, CROWS // SUB)],
                sems[NSTG + g],
            )
        )

    for c in outs:
        c.wait()


@jax.jit
def _embed(x, weight):
    mesh = plsc.VectorSubcoreMesh(core_axis_name="c", subcore_axis_name="s")
    fn = pl.kernel(
        _sc_body,
        out_type=jax.ShapeDtypeStruct((128, CROWS, D), jnp.float32),
        mesh=mesh,
        scratch_types=[
            pltpu.VMEM((128, 128), jnp.int32),
            pltpu.VMEM((NCHUNK, CROWS, D), jnp.float32),
        ]
        + [pltpu.SemaphoreType.DMA] * (2 * NCHUNK * SUB),
    )
    return fn(x, weight)


def kernel(x, weight):
    return _embed(x, weight)
